# baseline (speedup 1.0000x reference)
import functools

import jax
import jax.numpy as jnp
from jax import lax
from jax.experimental import pallas as pl
from jax.experimental.pallas import tpu as pltpu
from jax.experimental.pallas import tpu_sc as plsc

_INFO = plsc.get_sparse_core_info()
_NC = _INFO.num_cores
_NS = _INFO.num_subcores
_NW = _NC * _NS


@functools.lru_cache(maxsize=None)
def _make_gather(V, D, B, H):
    assert D == 2 * _NW
    BC = B // 128
    L = 16

    mesh = plsc.VectorSubcoreMesh(core_axis_name="c", subcore_axis_name="s")

    @functools.partial(
        pl.kernel,
        mesh=mesh,
        out_type=jax.ShapeDtypeStruct((H, D // 8, BC, 8, 128), jnp.float32),
        scratch_types=[
            pltpu.VMEM((V,), jnp.int32),
            pltpu.VMEM((2, BC, 128), jnp.int32),
            pltpu.VMEM((2, BC, 128), jnp.float32),
            pltpu.VMEM((2, BC, 128), jnp.float32),
            pltpu.SemaphoreType.DMA,
            pltpu.SemaphoreType.DMA,
        ],
        compiler_params=pltpu.CompilerParams(
            use_tc_tiling_on_sc=True, needs_layout_passes=False
        ),
    )
    def gather_kernel(idx_hbm, packed_hbm, out_hbm, tbl_v, idx_v, lrow_v,
                      hrow_v, osem, isem):
        cid = lax.axis_index("c")
        sid = lax.axis_index("s")
        worker = sid * _NC + cid

        e_lo = worker
        e_hi = worker + _NW
        erl, esl = e_lo // 8, lax.rem(e_lo, 8)
        erh, esh = e_hi // 8, lax.rem(e_hi, 8)

        def hh(h):
            return lax.rem(h + worker, H)

        def load_idx(h, slot):
            return pltpu.async_copy(idx_hbm.at[hh(h)], idx_v.at[slot], isem)

        i0 = load_idx(0, 0)
        i1 = load_idx(1, 1)
        pltpu.sync_copy(packed_hbm.at[worker], tbl_v)

        himask = jnp.full((L,), -65536, dtype=jnp.int32)

        def compute_row(slot):
            for r in range(0, BC, 2):
                ivs = [
                    idx_v[slot, r + q // 8, pl.ds((q % 8) * L, L)]
                    for q in range(16)
                ]
                packs = [plsc.load_gather(tbl_v, [iv]) for iv in ivs]
                for q in range(16):
                    pk = packs[q]
                    lo = plsc.bitcast(lax.shift_left(pk, 16), jnp.float32)
                    hi = plsc.bitcast(lax.bitwise_and(pk, himask), jnp.float32)
                    lrow_v[slot, r + q // 8, pl.ds((q % 8) * L, L)] = lo
                    hrow_v[slot, r + q // 8, pl.ds((q % 8) * L, L)] = hi

        def do_row(h, slot, prefetch):
            compute_row(slot)
            if prefetch is not None:
                load_idx(prefetch, slot)
            hr = hh(h)
            w1 = pltpu.async_copy(
                lrow_v.at[slot], out_hbm.at[hr, erl, slice(None), esl], osem
            )
            w2 = pltpu.async_copy(
                hrow_v.at[slot], out_hbm.at[hr, erh, slice(None), esh], osem
            )
            return w1, w2

        i0.wait()
        w0 = do_row(0, 0, 2)
        i1.wait()
        w1 = do_row(1, 1, 3)

        def body(t, carry):
            h = 2 * t + 2
            for d in w0:
                d.wait()
            i0.wait()
            do_row(h, 0, h + 2)
            for d in w1:
                d.wait()
            i1.wait()
            do_row(h + 1, 1, h + 3)
            return carry

        lax.fori_loop(0, (H - 2) // 2, body, 0)
        for d in (*w0, *w1):
            d.wait()
        i0.wait()
        i1.wait()

    return gather_kernel


def kernel(inputs, embedding_table):
    B, H = inputs.shape
    V, D = embedding_table.shape
    idx_t = jnp.transpose(inputs).astype(jnp.int32).reshape(H, B // 128, 128)
    tab_t = jnp.transpose(embedding_table)
    lo16 = lax.bitcast_convert_type(
        tab_t[: D // 2].astype(jnp.bfloat16), jnp.uint16
    ).astype(jnp.uint32)
    hi16 = lax.bitcast_convert_type(
        tab_t[D // 2 :].astype(jnp.bfloat16), jnp.uint16
    ).astype(jnp.uint32)
    packed = lax.bitcast_convert_type((hi16 << 16) | lo16, jnp.int32)
    out5 = _make_gather(V, D, B, H)(idx_t, packed)
    return jnp.transpose(out5, (2, 4, 0, 1, 3)).reshape(B, H, D)


# speedup vs baseline: 1.4894x; 1.4894x over previous
import functools

import jax
import jax.numpy as jnp
from jax import lax
from jax.experimental import pallas as pl
from jax.experimental.pallas import tpu as pltpu
from jax.experimental.pallas import tpu_sc as plsc

_INFO = plsc.get_sparse_core_info()
_NC = _INFO.num_cores
_NS = _INFO.num_subcores
_NW = _NC * _NS


@functools.lru_cache(maxsize=None)
def _make_gather(V, D, B, H):
    assert D % _NW == 0 or _NW % D == 0
    npass = D // _NW
    assert npass * _NW == D
    BC = B // 128
    L = 16

    mesh = plsc.VectorSubcoreMesh(core_axis_name="c", subcore_axis_name="s")

    @functools.partial(
        pl.kernel,
        mesh=mesh,
        out_type=jax.ShapeDtypeStruct((H, D // 8, BC, 8, 128), jnp.float32),
        scratch_types=[
            pltpu.VMEM((V,), jnp.float32),
            pltpu.VMEM((2, BC, 128), jnp.int32),
            pltpu.VMEM((2, BC, 128), jnp.float32),
            pltpu.VMEM_SHARED((H, BC, 128), jnp.int32),
            pltpu.SemaphoreType.DMA,
            pltpu.SemaphoreType.DMA,
        ],
        compiler_params=pltpu.CompilerParams(
            use_tc_tiling_on_sc=True, needs_layout_passes=False
        ),
    )
    def gather_kernel(idx_hbm, table_hbm, out_hbm, tbl_v, idx_v, row_v,
                      idx_sh, osem, isem):
        cid = lax.axis_index("c")
        sid = lax.axis_index("s")
        worker = sid * _NC + cid

        @pl.when(sid == 0)
        def _():
            pltpu.sync_copy(idx_hbm, idx_sh)

        plsc.subcore_barrier()

        def load_idx(h, slot):
            return pltpu.async_copy(idx_sh.at[h], idx_v.at[slot], isem)

        i0 = load_idx(0, 0)
        i1 = load_idx(1, 1)

        def compute_row(slot):
            for r in range(0, BC, 2):
                ivs = [
                    idx_v[slot, r + q // 8, pl.ds((q % 8) * L, L)]
                    for q in range(16)
                ]
                vals = [plsc.load_gather(tbl_v, [iv]) for iv in ivs]
                for q in range(16):
                    row_v[slot, r + q // 8, pl.ds((q % 8) * L, L)] = vals[q]

        for p in range(npass):
            e = p * _NW + worker
            er = e // 8
            es = lax.rem(e, 8)
            pltpu.sync_copy(table_hbm.at[e], tbl_v)

            def do_row(h, slot):
                compute_row(slot)
                load_idx(lax.rem(h + 2, H), slot)
                return pltpu.async_copy(
                    row_v.at[slot], out_hbm.at[h, er, slice(None), es], osem
                )

            i0.wait()
            w0 = do_row(0, 0)
            i1.wait()
            w1 = do_row(1, 1)

            def body(t, carry):
                h = 2 * t + 2
                w0.wait()
                i0.wait()
                do_row(h, 0)
                w1.wait()
                i1.wait()
                do_row(h + 1, 1)
                return carry

            lax.fori_loop(0, (H - 2) // 2, body, 0)
            w0.wait()
            w1.wait()

        i0.wait()
        i1.wait()

    return gather_kernel


def kernel(inputs, embedding_table):
    B, H = inputs.shape
    V, D = embedding_table.shape
    idx_t = jnp.transpose(inputs).astype(jnp.int32).reshape(H, B // 128, 128)
    tab_t = jnp.transpose(embedding_table)
    out5 = _make_gather(V, D, B, H)(idx_t, tab_t)
    return jnp.transpose(out5, (2, 4, 0, 1, 3)).reshape(B, H, D)


# speedup vs baseline: 1.4955x; 1.0041x over previous
import functools

import jax
import jax.numpy as jnp
from jax import lax
from jax.experimental import pallas as pl
from jax.experimental.pallas import tpu as pltpu
from jax.experimental.pallas import tpu_sc as plsc

_INFO = plsc.get_sparse_core_info()
_NC = _INFO.num_cores
_NS = _INFO.num_subcores
_NW = _NC * _NS


@functools.lru_cache(maxsize=None)
def _make_gather(V, D, B, H):
    assert D % _NW == 0 or _NW % D == 0
    npass = D // _NW
    assert npass * _NW == D
    BC = B // 128
    L = 16

    mesh = plsc.VectorSubcoreMesh(core_axis_name="c", subcore_axis_name="s")

    @functools.partial(
        pl.kernel,
        mesh=mesh,
        out_type=jax.ShapeDtypeStruct((H, D // 8, BC, 8, 128), jnp.float32),
        scratch_types=[
            pltpu.VMEM((V,), jnp.float32),
            pltpu.VMEM((2, BC, 128), jnp.int32),
            pltpu.VMEM((2, BC, 128), jnp.float32),
            pltpu.VMEM_SHARED((H, BC, 128), jnp.int32),
            pltpu.SemaphoreType.DMA,
            pltpu.SemaphoreType.DMA,
        ],
        compiler_params=pltpu.CompilerParams(
            use_tc_tiling_on_sc=True, needs_layout_passes=False
        ),
    )
    def gather_kernel(idx_hbm, table_hbm, out_hbm, tbl_v, idx_v, row_v,
                      idx_sh, osem, isem):
        cid = lax.axis_index("c")
        sid = lax.axis_index("s")
        worker = sid * _NC + cid

        @pl.when(sid == 0)
        def _():
            pltpu.sync_copy(idx_hbm, idx_sh)

        plsc.subcore_barrier()

        def load_idx(h, slot):
            return pltpu.async_copy(
                idx_sh.at[lax.rem(h + 3 * sid, H)], idx_v.at[slot], isem
            )

        i0 = load_idx(0, 0)
        i1 = load_idx(1, 1)

        def compute_row(slot):
            for r in range(0, BC, 2):
                ivs = [
                    idx_v[slot, r + q // 8, pl.ds((q % 8) * L, L)]
                    for q in range(16)
                ]
                vals = [plsc.load_gather(tbl_v, [iv]) for iv in ivs]
                for q in range(16):
                    row_v[slot, r + q // 8, pl.ds((q % 8) * L, L)] = vals[q]

        for p in range(npass):
            e = p * _NW + worker
            er = e // 8
            es = lax.rem(e, 8)
            pltpu.sync_copy(table_hbm.at[e], tbl_v)

            def do_row(h, slot):
                compute_row(slot)
                load_idx(lax.rem(h + 2, H), slot)
                hr = lax.rem(h + 3 * sid, H)
                return pltpu.async_copy(
                    row_v.at[slot], out_hbm.at[hr, er, slice(None), es], osem
                )

            i0.wait()
            w0 = do_row(0, 0)
            i1.wait()
            w1 = do_row(1, 1)

            def body(t, carry):
                h = 2 * t + 2
                w0.wait()
                i0.wait()
                do_row(h, 0)
                w1.wait()
                i1.wait()
                do_row(h + 1, 1)
                return carry

            lax.fori_loop(0, (H - 2) // 2, body, 0)
            w0.wait()
            w1.wait()

        i0.wait()
        i1.wait()

    return gather_kernel


def kernel(inputs, embedding_table):
    B, H = inputs.shape
    V, D = embedding_table.shape
    idx_t = jnp.transpose(inputs).astype(jnp.int32).reshape(H, B // 128, 128)
    tab_t = jnp.transpose(embedding_table)
    out5 = _make_gather(V, D, B, H)(idx_t, tab_t)
    return jnp.transpose(out5, (2, 4, 0, 1, 3)).reshape(B, H, D)
